# Initial kernel scaffold; baseline (speedup 1.0000x reference)
import jax
import jax.numpy as jnp
from jax.experimental import pallas as pl


def kernel(x, batch, a1_p, a2_p, a3_p):
    raise NotImplementedError("write your pallas kernel here")


# speedup vs baseline: 32.3198x; 32.3198x over previous
import numpy as np
import jax
import jax.numpy as jnp
from jax import lax
from jax.experimental import pallas as pl
from jax.experimental.pallas import tpu as pltpu

_NUM_SEG = 16
_EPS = 1e-08


def _uot_body(a_ref, x_ref, brow_ref, z_ref, y_ref):
    x = x_ref[...]
    brow = brow_ref[...]
    n, d = x.shape
    B = z_ref.shape[0]
    K = a_ref.shape[1]
    f32 = jnp.float32

    a = a_ref[...]
    sp = jnp.maximum(a, 0.0) + jnp.log(1.0 + jnp.exp(-jnp.abs(a)))

    onehot_t = (brow == lax.broadcasted_iota(jnp.int32, (B, n), 0)).astype(f32)

    def seg_sum(m):
        return lax.dot_general(onehot_t, m, (((1,), (0,)), ((), ())),
                               preferred_element_type=f32)

    def _bcast1(m):
        return lax.dot_general(onehot_t, m, (((0,), (0,)), ((), ())),
                               preferred_element_type=f32)

    def seg_bcast(m):
        m_hi = m.astype(jnp.bfloat16).astype(f32)
        return _bcast1(m_hi) + _bcast1(m - m_hi)

    counts = jnp.sum(onehot_t, axis=1, keepdims=True)
    nonempty = counts > 0.0
    nlc = jnp.where(nonempty, -jnp.log(jnp.maximum(counts, 1.0)), 0.0)
    nlc_b = jnp.broadcast_to(nlc, (B, d))
    log_u2 = -float(np.log(float(d)))

    b2 = jnp.zeros((B, d), f32)
    inva1_prev = 1.0 / sp[0:1, 0:1]
    y = x * inva1_prev
    for k in range(K):
        a1 = sp[0:1, k:k + 1]
        a2 = sp[1:2, k:k + 1]
        a3 = sp[2:3, k:k + 1]
        inva1 = 1.0 / a1
        c1a = a2 * inva1 / (a1 + a2)
        c1b = a2 / (a1 + a2)
        c2a = a3 * inva1 / (a1 + a3)
        c2b = a3 / (a1 + a3)

        g = jnp.max(jnp.max(y, axis=1, keepdims=True), axis=0, keepdims=True)
        e = jnp.exp(y - g)
        s1 = jnp.sum(e, axis=1, keepdims=True)
        log_mu1 = jnp.log(s1) + g
        seg = seg_sum(e)
        log_mu2 = jnp.log(seg) + g

        b2_new = c2a * b2 + c2b * (log_u2 - log_mu2)
        b2_new = jnp.where(nonempty, b2_new, 0.0)
        m_seg = b2_new - c1a * b2 + c1b * nlc_b
        y = (x * (inva1 - c1a * inva1_prev) + c1a * y
             - c1b * log_mu1 + seg_bcast(m_seg))
        b2 = b2_new
        inva1_prev = inva1

    ey = jnp.exp(y) + _EPS
    y_ref[...] = ey
    z_ref[...] = float(d) * seg_sum(x * ey)


def kernel(x, batch, a1_p, a2_p, a3_p):
    n, d = x.shape
    B = _NUM_SEG
    a = jnp.stack([a1_p, a2_p, a3_p]).astype(jnp.float32)
    brow = batch.astype(jnp.int32).reshape(1, n)
    z, y = pl.pallas_call(
        _uot_body,
        out_shape=(
            jax.ShapeDtypeStruct((B, d), jnp.float32),
            jax.ShapeDtypeStruct((n, d), jnp.float32),
        ),
    )(a, x, brow)
    return (z, y)
